# baseline (speedup 1.0000x reference)
import functools

import jax
import jax.numpy as jnp
import numpy as np
from jax import lax
from jax.experimental import pallas as pl
from jax.experimental.pallas import tpu as pltpu
from jax.experimental.pallas import tpu_sc as plsc

NC = 2
NS = 16
L = 16
CHUNK = 2048

_PHI_C = np.abs(np.array([0.18175, 0.50986, 0.28022, 0.02817], np.float32))
_PHI_E = np.abs(np.array([3.1998, 0.94229, 0.4029, 0.20162], np.float32))
_SOFT = np.exp(_PHI_C - np.max(_PHI_C))
_COEF = (_SOFT / np.sum(_SOFT)).astype(np.float32)
_AEXP = (_PHI_E - _PHI_E[3]).astype(np.float32)
_INV_A = np.float32(1.0) / np.float32(0.8854)


def _zbl_body(znp_hbm, dx_hbm, dy_hbm, dz_hbm, ii_hbm, ij_hbm, zat_hbm,
              zero_hbm, out_hbm, ztab, zatab, iibA, ijbA, dxbA, dybA, dzbA,
              iibB, ijbB, dxbB, dybB, dzbB, repb, acc, insem,
              n_nodes, n_edges):
    cid = lax.axis_index("c")
    sid = lax.axis_index("s")
    wid = cid * NS + sid
    ept = n_edges // (NC * NS)
    nfull = ept // CHUNK
    pipe = nfull - (nfull % 2)
    tail = ept - nfull * CHUNK
    base0 = wid * ept

    bufsA = (iibA, ijbA, dxbA, dybA, dzbA)
    bufsB = (iibB, ijbB, dxbB, dybB, dzbB)
    hbms = (ii_hbm, ij_hbm, dx_hbm, dy_hbm, dz_hbm)

    pltpu.sync_copy(znp_hbm, ztab)
    pltpu.sync_copy(zat_hbm, zatab)

    @pl.when(sid == 0)
    def _():
        pltpu.sync_copy(zero_hbm, acc)

    plsc.subcore_barrier()

    def input_copies(c, bufs, semidx):
        base = base0 + c * CHUNK
        return [
            pltpu.make_async_copy(src.at[pl.ds(base, CHUNK)], dst,
                                  insem.at[semidx])
            for src, dst in zip(hbms, bufs)
        ]

    def compute_vec(j, ii, ij, dx, dy, dz):
        wi = plsc.load_gather(ztab, [lax.shift_right_logical(ii, 1)])
        wj = plsc.load_gather(ztab, [lax.shift_right_logical(ij, 1)])
        shi = lax.shift_left(jnp.bitwise_and(ii, 1), 4)
        shj = lax.shift_left(jnp.bitwise_and(ij, 1), 4)
        zi = jnp.bitwise_and(lax.shift_right_logical(wi, shi),
                             jnp.int32(0xFFFF))
        zj = jnp.bitwise_and(lax.shift_right_logical(wj, shj),
                             jnp.int32(0xFFFF))
        zai = plsc.load_gather(zatab, [zi])
        zaj = plsc.load_gather(zatab, [zj])
        d2 = jnp.maximum(dx * dx + dy * dy + dz * dz, jnp.float32(1e-20))
        bits = lax.bitcast_convert_type(d2, jnp.int32)
        y = lax.bitcast_convert_type(
            jnp.int32(0x5F3759DF) - lax.shift_right_arithmetic(bits, 1),
            jnp.float32)
        half = jnp.float32(0.5) * d2
        for _ in range(3):
            y = y * (jnp.float32(1.5) - half * y * y)
        dist = d2 * y
        arg = dist * (zai + zaj) * _INV_A
        phi = (_COEF[0] * jnp.exp(-_AEXP[0] * arg)
               + _COEF[1] * jnp.exp(-_AEXP[1] * arg)
               + _COEF[2] * jnp.exp(-_AEXP[2] * arg)
               + _COEF[3])
        x = jnp.float32(5.0) - dist
        sw = ((jnp.float32(6.0) * x - jnp.float32(15.0)) * x
              + jnp.float32(10.0)) * x * x * x
        sw = jnp.where(dist < jnp.float32(4.0), jnp.float32(1.0),
                       jnp.where(dist >= jnp.float32(5.0), jnp.float32(0.0),
                                 sw))
        sw = jnp.maximum(sw, jnp.float32(1e-30))
        zif = zi.astype(jnp.float32)
        zjf = zj.astype(jnp.float32)
        rep = (jnp.float32(0.5) * zif * zjf) * phi * sw * y
        repb[pl.ds(j * L, L)] = rep

    def compute_and_scatter(bufs, nvec):
        iil, ijl, dxl, dyl, dzl = bufs

        def vec_body(j, carry2):
            b16 = j * L
            compute_vec(j, iil[pl.ds(b16, L)], ijl[pl.ds(b16, L)],
                        dxl[pl.ds(b16, L)], dyl[pl.ds(b16, L)],
                        dzl[pl.ds(b16, L)])
            return carry2

        lax.fori_loop(0, nvec, vec_body, 0, unroll=4)
        pltpu.sync_copy(repb, acc.at[iil], add=True)

    if pipe:
        for cp in input_copies(0, bufsA, 0):
            cp.start()
        for cp in input_copies(1, bufsB, 1):
            cp.start()

        def pair_body(p, carry):
            c0 = 2 * p
            for cp in input_copies(c0, bufsA, 0):
                cp.wait()
            compute_and_scatter(bufsA, CHUNK // L)

            @pl.when(c0 + 2 < pipe)
            def _():
                for cp in input_copies(c0 + 2, bufsA, 0):
                    cp.start()

            for cp in input_copies(c0 + 1, bufsB, 1):
                cp.wait()
            compute_and_scatter(bufsB, CHUNK // L)

            @pl.when(c0 + 3 < pipe)
            def _():
                for cp in input_copies(c0 + 3, bufsB, 1):
                    cp.start()

            return carry

        lax.fori_loop(0, pipe // 2, pair_body, 0)

    for c in range(pipe, nfull):
        base = base0 + c * CHUNK
        for src, dst in zip(hbms, bufsA):
            pltpu.sync_copy(src.at[pl.ds(base, CHUNK)], dst)
        compute_and_scatter(bufsA, CHUNK // L)

    if tail:
        base = base0 + nfull * CHUNK
        for src, dst in zip(hbms, bufsA):
            pltpu.sync_copy(src.at[pl.ds(base, tail)],
                            dst.at[pl.ds(0, tail)])

        def tvec_body(j, carry2):
            b16 = j * L
            compute_vec(j, iibA[pl.ds(b16, L)], ijbA[pl.ds(b16, L)],
                        dxbA[pl.ds(b16, L)], dybA[pl.ds(b16, L)],
                        dzbA[pl.ds(b16, L)])
            return carry2

        lax.fori_loop(0, tail // L, tvec_body, 0)
        zf = jnp.zeros((L,), jnp.float32)
        zidx = jnp.zeros((L,), jnp.int32)
        for t in range((CHUNK - tail) // L):
            off = tail + t * L
            repb[pl.ds(off, L)] = zf
            iibA[pl.ds(off, L)] = zidx
        pltpu.sync_copy(repb, acc.at[iibA], add=True)

    plsc.subcore_barrier()

    @pl.when(sid == 0)
    def _():
        pltpu.sync_copy(acc, out_hbm.at[cid])


def kernel(atomic_numbers, displacements, idx_i, idx_j, atom_mask,
           batch_segments, batch_mask, batch_size):
    n_nodes = atomic_numbers.shape[0]
    n_edges = idx_i.shape[0]
    zn = atomic_numbers.astype(jnp.int32)
    znp = zn[0::2] | (zn[1::2] << 16)
    dx = displacements[:, 0]
    dy = displacements[:, 1]
    dz = displacements[:, 2]
    zat = jnp.power(jnp.arange(256, dtype=jnp.float32), jnp.float32(0.23))
    zeros_nodes = jnp.zeros((n_nodes,), jnp.float32)

    edge_buf = lambda dt: pltpu.VMEM((CHUNK,), dt)
    body = functools.partial(_zbl_body, n_nodes=n_nodes, n_edges=n_edges)
    run = pl.kernel(
        body,
        mesh=plsc.VectorSubcoreMesh(core_axis_name="c", subcore_axis_name="s"),
        out_type=jax.ShapeDtypeStruct((NC, n_nodes), jnp.float32),
        compiler_params=pltpu.CompilerParams(needs_layout_passes=False),
        scratch_types=[
            pltpu.VMEM((n_nodes // 2,), jnp.int32),
            pltpu.VMEM((256,), jnp.float32),
            edge_buf(jnp.int32),
            edge_buf(jnp.int32),
            edge_buf(jnp.float32),
            edge_buf(jnp.float32),
            edge_buf(jnp.float32),
            edge_buf(jnp.int32),
            edge_buf(jnp.int32),
            edge_buf(jnp.float32),
            edge_buf(jnp.float32),
            edge_buf(jnp.float32),
            edge_buf(jnp.float32),
            pltpu.VMEM_SHARED((n_nodes,), jnp.float32),
            pltpu.SemaphoreType.DMA((2,)),
        ],
    )
    partial = run(znp, dx, dy, dz, idx_i.astype(jnp.int32),
                  idx_j.astype(jnp.int32), zat, zeros_nodes)
    erep = (partial[0] + partial[1]) * atom_mask
    return erep[..., None, None, None]


# speedup vs baseline: 1.0813x; 1.0813x over previous
import functools

import jax
import jax.numpy as jnp
import numpy as np
from jax import lax
from jax.experimental import pallas as pl
from jax.experimental.pallas import tpu as pltpu
from jax.experimental.pallas import tpu_sc as plsc

NC = 2
NS = 16
L = 16
CHUNK = 4096

_PHI_C = np.abs(np.array([0.18175, 0.50986, 0.28022, 0.02817], np.float32))
_PHI_E = np.abs(np.array([3.1998, 0.94229, 0.4029, 0.20162], np.float32))
_SOFT = np.exp(_PHI_C - np.max(_PHI_C))
_COEF = (_SOFT / np.sum(_SOFT)).astype(np.float32)
_AEXP = (_PHI_E - _PHI_E[3]).astype(np.float32)
_INV_A = np.float32(1.0) / np.float32(0.8854)


def _zbl_body(znp_hbm, dx_hbm, dy_hbm, dz_hbm, ii_hbm, ij_hbm, zat_hbm,
              zero_hbm, out_hbm, ztab, zatab, iibA, ijbA, dxbA, dybA, dzbA,
              iibB, ijbB, dxbB, dybB, dzbB, repb, acc, insem,
              n_nodes, n_edges):
    cid = lax.axis_index("c")
    sid = lax.axis_index("s")
    wid = cid * NS + sid
    ept = n_edges // (NC * NS)
    nfull = ept // CHUNK
    pipe = nfull - (nfull % 2)
    tail = ept - nfull * CHUNK
    base0 = wid * ept

    bufsA = (iibA, ijbA, dxbA, dybA, dzbA)
    bufsB = (iibB, ijbB, dxbB, dybB, dzbB)
    hbms = (ii_hbm, ij_hbm, dx_hbm, dy_hbm, dz_hbm)

    pltpu.sync_copy(znp_hbm, ztab)
    pltpu.sync_copy(zat_hbm, zatab)

    @pl.when(sid == 0)
    def _():
        pltpu.sync_copy(zero_hbm, acc)

    plsc.subcore_barrier()

    def input_copies(c, bufs, semidx):
        base = base0 + c * CHUNK
        return [
            pltpu.make_async_copy(src.at[pl.ds(base, CHUNK)], dst,
                                  insem.at[semidx])
            for src, dst in zip(hbms, bufs)
        ]

    def compute_vec(j, ii, ij, dx, dy, dz):
        wi = plsc.load_gather(ztab, [lax.shift_right_logical(ii, 1)])
        wj = plsc.load_gather(ztab, [lax.shift_right_logical(ij, 1)])
        shi = lax.shift_left(jnp.bitwise_and(ii, 1), 4)
        shj = lax.shift_left(jnp.bitwise_and(ij, 1), 4)
        zi = jnp.bitwise_and(lax.shift_right_logical(wi, shi),
                             jnp.int32(0xFFFF))
        zj = jnp.bitwise_and(lax.shift_right_logical(wj, shj),
                             jnp.int32(0xFFFF))
        zai = plsc.load_gather(zatab, [zi])
        zaj = plsc.load_gather(zatab, [zj])
        d2 = jnp.maximum(dx * dx + dy * dy + dz * dz, jnp.float32(1e-20))
        bits = lax.bitcast_convert_type(d2, jnp.int32)
        y = lax.bitcast_convert_type(
            jnp.int32(0x5F3759DF) - lax.shift_right_arithmetic(bits, 1),
            jnp.float32)
        half = jnp.float32(0.5) * d2
        for _ in range(3):
            y = y * (jnp.float32(1.5) - half * y * y)
        dist = d2 * y
        arg = dist * (zai + zaj) * _INV_A
        phi = (_COEF[0] * jnp.exp(-_AEXP[0] * arg)
               + _COEF[1] * jnp.exp(-_AEXP[1] * arg)
               + _COEF[2] * jnp.exp(-_AEXP[2] * arg)
               + _COEF[3])
        x = jnp.float32(5.0) - dist
        sw = ((jnp.float32(6.0) * x - jnp.float32(15.0)) * x
              + jnp.float32(10.0)) * x * x * x
        sw = jnp.where(dist < jnp.float32(4.0), jnp.float32(1.0),
                       jnp.where(dist >= jnp.float32(5.0), jnp.float32(0.0),
                                 sw))
        sw = jnp.maximum(sw, jnp.float32(1e-30))
        zif = zi.astype(jnp.float32)
        zjf = zj.astype(jnp.float32)
        rep = (jnp.float32(0.5) * zif * zjf) * phi * sw * y
        repb[pl.ds(j * L, L)] = rep

    def compute_and_scatter(bufs, nvec):
        iil, ijl, dxl, dyl, dzl = bufs

        def vec_body(j, carry2):
            b16 = j * L
            compute_vec(j, iil[pl.ds(b16, L)], ijl[pl.ds(b16, L)],
                        dxl[pl.ds(b16, L)], dyl[pl.ds(b16, L)],
                        dzl[pl.ds(b16, L)])
            return carry2

        lax.fori_loop(0, nvec, vec_body, 0)
        pltpu.sync_copy(repb, acc.at[iil], add=True)

    if pipe:
        for cp in input_copies(0, bufsA, 0):
            cp.start()
        for cp in input_copies(1, bufsB, 1):
            cp.start()

        def pair_body(p, carry):
            c0 = 2 * p
            for cp in input_copies(c0, bufsA, 0):
                cp.wait()
            compute_and_scatter(bufsA, CHUNK // L)

            @pl.when(c0 + 2 < pipe)
            def _():
                for cp in input_copies(c0 + 2, bufsA, 0):
                    cp.start()

            for cp in input_copies(c0 + 1, bufsB, 1):
                cp.wait()
            compute_and_scatter(bufsB, CHUNK // L)

            @pl.when(c0 + 3 < pipe)
            def _():
                for cp in input_copies(c0 + 3, bufsB, 1):
                    cp.start()

            return carry

        lax.fori_loop(0, pipe // 2, pair_body, 0)

    for c in range(pipe, nfull):
        base = base0 + c * CHUNK
        for src, dst in zip(hbms, bufsA):
            pltpu.sync_copy(src.at[pl.ds(base, CHUNK)], dst)
        compute_and_scatter(bufsA, CHUNK // L)

    if tail:
        base = base0 + nfull * CHUNK
        for src, dst in zip(hbms, bufsA):
            pltpu.sync_copy(src.at[pl.ds(base, tail)],
                            dst.at[pl.ds(0, tail)])

        def tvec_body(j, carry2):
            b16 = j * L
            compute_vec(j, iibA[pl.ds(b16, L)], ijbA[pl.ds(b16, L)],
                        dxbA[pl.ds(b16, L)], dybA[pl.ds(b16, L)],
                        dzbA[pl.ds(b16, L)])
            return carry2

        lax.fori_loop(0, tail // L, tvec_body, 0)
        zf = jnp.zeros((L,), jnp.float32)
        zidx = jnp.zeros((L,), jnp.int32)
        for t in range((CHUNK - tail) // L):
            off = tail + t * L
            repb[pl.ds(off, L)] = zf
            iibA[pl.ds(off, L)] = zidx
        pltpu.sync_copy(repb, acc.at[iibA], add=True)

    plsc.subcore_barrier()

    @pl.when(sid == 0)
    def _():
        pltpu.sync_copy(acc, out_hbm.at[cid])


def kernel(atomic_numbers, displacements, idx_i, idx_j, atom_mask,
           batch_segments, batch_mask, batch_size):
    n_nodes = atomic_numbers.shape[0]
    n_edges = idx_i.shape[0]
    zn = atomic_numbers.astype(jnp.int32)
    znp = zn[0::2] | (zn[1::2] << 16)
    dx = displacements[:, 0]
    dy = displacements[:, 1]
    dz = displacements[:, 2]
    zat = jnp.power(jnp.arange(256, dtype=jnp.float32), jnp.float32(0.23))
    zeros_nodes = jnp.zeros((n_nodes,), jnp.float32)

    edge_buf = lambda dt: pltpu.VMEM((CHUNK,), dt)
    body = functools.partial(_zbl_body, n_nodes=n_nodes, n_edges=n_edges)
    run = pl.kernel(
        body,
        mesh=plsc.VectorSubcoreMesh(core_axis_name="c", subcore_axis_name="s"),
        out_type=jax.ShapeDtypeStruct((NC, n_nodes), jnp.float32),
        compiler_params=pltpu.CompilerParams(needs_layout_passes=False),
        scratch_types=[
            pltpu.VMEM((n_nodes // 2,), jnp.int32),
            pltpu.VMEM((256,), jnp.float32),
            edge_buf(jnp.int32),
            edge_buf(jnp.int32),
            edge_buf(jnp.float32),
            edge_buf(jnp.float32),
            edge_buf(jnp.float32),
            edge_buf(jnp.int32),
            edge_buf(jnp.int32),
            edge_buf(jnp.float32),
            edge_buf(jnp.float32),
            edge_buf(jnp.float32),
            edge_buf(jnp.float32),
            pltpu.VMEM_SHARED((n_nodes,), jnp.float32),
            pltpu.SemaphoreType.DMA((2,)),
        ],
    )
    partial = run(znp, dx, dy, dz, idx_i.astype(jnp.int32),
                  idx_j.astype(jnp.int32), zat, zeros_nodes)
    erep = (partial[0] + partial[1]) * atom_mask
    return erep[..., None, None, None]


# speedup vs baseline: 1.2834x; 1.1869x over previous
import functools

import jax
import jax.numpy as jnp
import numpy as np
from jax import lax
from jax.experimental import pallas as pl
from jax.experimental.pallas import tpu as pltpu
from jax.experimental.pallas import tpu_sc as plsc

NC = 2
NS = 16
L = 16
CHUNK = 2048
TCBLK = 128000

_PHI_C = np.abs(np.array([0.18175, 0.50986, 0.28022, 0.02817], np.float32))
_PHI_E = np.abs(np.array([3.1998, 0.94229, 0.4029, 0.20162], np.float32))
_SOFT = np.exp(_PHI_C - np.max(_PHI_C))
_COEF = (_SOFT / np.sum(_SOFT)).astype(np.float32)
_AEXP = (_PHI_E - _PHI_E[3]).astype(np.float32)
_ZSCALE = np.float32(32768.0)


def _dist_tc_kernel(dx_ref, dy_ref, dz_ref, dist_ref, t_ref):
    dx = dx_ref[...]
    dy = dy_ref[...]
    dz = dz_ref[...]
    d2 = jnp.maximum(dx * dx + dy * dy + dz * dz, jnp.float32(1e-20))
    r = lax.rsqrt(d2)
    dist = d2 * r
    x = jnp.float32(5.0) - dist
    sw = ((jnp.float32(6.0) * x - jnp.float32(15.0)) * x
          + jnp.float32(10.0)) * x * x * x
    sw = jnp.where(dist < jnp.float32(4.0), jnp.float32(1.0),
                   jnp.where(dist >= jnp.float32(5.0), jnp.float32(0.0),
                             sw))
    sw = jnp.maximum(sw, jnp.float32(1e-30))
    dist_ref[...] = dist
    t_ref[...] = jnp.float32(0.5) * sw * r


def _edge_dist_t(dx, dy, dz):
    n_edges = dx.shape[0]
    spec = pl.BlockSpec((TCBLK,), lambda i: (i,))
    return pl.pallas_call(
        _dist_tc_kernel,
        grid=(n_edges // TCBLK,),
        in_specs=[spec, spec, spec],
        out_specs=[spec, spec],
        out_shape=[jax.ShapeDtypeStruct((n_edges,), jnp.float32)] * 2,
    )(dx, dy, dz)


def _zbl_body(ztab_hbm, dist_hbm, t_hbm, ii_hbm, ij_hbm, zero_hbm, out_hbm,
              ztab, iibA, ijbA, dsbA, tbA, iibB, ijbB, dsbB, tbB,
              repb, acc, insem, n_nodes, n_edges):
    cid = lax.axis_index("c")
    sid = lax.axis_index("s")
    wid = cid * NS + sid
    ept = n_edges // (NC * NS)
    nfull = ept // CHUNK
    pipe = nfull - (nfull % 2)
    tail = ept - nfull * CHUNK
    base0 = wid * ept

    bufsA = (iibA, ijbA, dsbA, tbA)
    bufsB = (iibB, ijbB, dsbB, tbB)
    hbms = (ii_hbm, ij_hbm, dist_hbm, t_hbm)

    pltpu.sync_copy(ztab_hbm, ztab)

    @pl.when(sid == 0)
    def _():
        pltpu.sync_copy(zero_hbm, acc)

    plsc.subcore_barrier()

    def input_copies(c, bufs, semidx):
        base = base0 + c * CHUNK
        return [
            pltpu.make_async_copy(src.at[pl.ds(base, CHUNK)], dst,
                                  insem.at[semidx])
            for src, dst in zip(hbms, bufs)
        ]

    def compute_vec(j, ii, ij, dist, t):
        wi = plsc.load_gather(ztab, [ii])
        wj = plsc.load_gather(ztab, [ij])
        mask = jnp.int32(0x1FFFF)
        zsci = jnp.bitwise_and(wi, mask).astype(jnp.float32)
        zscj = jnp.bitwise_and(wj, mask).astype(jnp.float32)
        zif = lax.shift_right_logical(wi, 17).astype(jnp.float32)
        zjf = lax.shift_right_logical(wj, 17).astype(jnp.float32)
        arg = dist * (zsci + zscj) * (jnp.float32(1.0) / _ZSCALE)
        phi = (_COEF[0] * jnp.exp(-_AEXP[0] * arg)
               + _COEF[1] * jnp.exp(-_AEXP[1] * arg)
               + _COEF[2] * jnp.exp(-_AEXP[2] * arg)
               + _COEF[3])
        rep = (zif * zjf) * phi * t
        repb[pl.ds(j * L, L)] = rep

    def compute_and_scatter(bufs, nvec):
        iil, ijl, dsl, tl = bufs

        def vec_body(j, carry2):
            b16 = j * L
            compute_vec(j, iil[pl.ds(b16, L)], ijl[pl.ds(b16, L)],
                        dsl[pl.ds(b16, L)], tl[pl.ds(b16, L)])
            return carry2

        lax.fori_loop(0, nvec, vec_body, 0)
        pltpu.sync_copy(repb, acc.at[iil], add=True)

    if pipe:
        for cp in input_copies(0, bufsA, 0):
            cp.start()
        for cp in input_copies(1, bufsB, 1):
            cp.start()

        def pair_body(p, carry):
            c0 = 2 * p
            for cp in input_copies(c0, bufsA, 0):
                cp.wait()
            compute_and_scatter(bufsA, CHUNK // L)

            @pl.when(c0 + 2 < pipe)
            def _():
                for cp in input_copies(c0 + 2, bufsA, 0):
                    cp.start()

            for cp in input_copies(c0 + 1, bufsB, 1):
                cp.wait()
            compute_and_scatter(bufsB, CHUNK // L)

            @pl.when(c0 + 3 < pipe)
            def _():
                for cp in input_copies(c0 + 3, bufsB, 1):
                    cp.start()

            return carry

        lax.fori_loop(0, pipe // 2, pair_body, 0)

    for c in range(pipe, nfull):
        base = base0 + c * CHUNK
        for src, dst in zip(hbms, bufsA):
            pltpu.sync_copy(src.at[pl.ds(base, CHUNK)], dst)
        compute_and_scatter(bufsA, CHUNK // L)

    if tail:
        base = base0 + nfull * CHUNK
        for src, dst in zip(hbms, bufsA):
            pltpu.sync_copy(src.at[pl.ds(base, tail)],
                            dst.at[pl.ds(0, tail)])

        def tvec_body(j, carry2):
            b16 = j * L
            compute_vec(j, iibA[pl.ds(b16, L)], ijbA[pl.ds(b16, L)],
                        dsbA[pl.ds(b16, L)], tbA[pl.ds(b16, L)])
            return carry2

        lax.fori_loop(0, tail // L, tvec_body, 0)
        zf = jnp.zeros((L,), jnp.float32)
        zidx = jnp.zeros((L,), jnp.int32)
        for t in range((CHUNK - tail) // L):
            off = tail + t * L
            repb[pl.ds(off, L)] = zf
            iibA[pl.ds(off, L)] = zidx
        pltpu.sync_copy(repb, acc.at[iibA], add=True)

    plsc.subcore_barrier()

    @pl.when(sid == 0)
    def _():
        pltpu.sync_copy(acc, out_hbm.at[cid])


def kernel(atomic_numbers, displacements, idx_i, idx_j, atom_mask,
           batch_segments, batch_mask, batch_size):
    n_nodes = atomic_numbers.shape[0]
    n_edges = idx_i.shape[0]
    zn = atomic_numbers.astype(jnp.int32)
    za = jnp.power(zn.astype(jnp.float32), jnp.float32(0.23))
    zfix = jnp.round(za * (_ZSCALE / jnp.float32(0.8854))).astype(jnp.int32)
    ztab = (zn << 17) | zfix
    dx = displacements[:, 0]
    dy = displacements[:, 1]
    dz = displacements[:, 2]
    dist, t = _edge_dist_t(dx, dy, dz)
    zeros_nodes = jnp.zeros((n_nodes,), jnp.float32)

    edge_buf = lambda dt: pltpu.VMEM((CHUNK,), dt)
    body = functools.partial(_zbl_body, n_nodes=n_nodes, n_edges=n_edges)
    run = pl.kernel(
        body,
        mesh=plsc.VectorSubcoreMesh(core_axis_name="c", subcore_axis_name="s"),
        out_type=jax.ShapeDtypeStruct((NC, n_nodes), jnp.float32),
        compiler_params=pltpu.CompilerParams(needs_layout_passes=False),
        scratch_types=[
            pltpu.VMEM((n_nodes,), jnp.int32),
            edge_buf(jnp.int32),
            edge_buf(jnp.int32),
            edge_buf(jnp.float32),
            edge_buf(jnp.float32),
            edge_buf(jnp.int32),
            edge_buf(jnp.int32),
            edge_buf(jnp.float32),
            edge_buf(jnp.float32),
            edge_buf(jnp.float32),
            pltpu.VMEM_SHARED((n_nodes,), jnp.float32),
            pltpu.SemaphoreType.DMA((2,)),
        ],
    )
    partial = run(ztab, dist, t, idx_i.astype(jnp.int32),
                  idx_j.astype(jnp.int32), zeros_nodes)
    erep = (partial[0] + partial[1]) * atom_mask
    return erep[..., None, None, None]
